# baseline (speedup 1.0000x reference)
import jax
import jax.numpy as jnp
from jax.experimental import pallas as pl
from jax.experimental.pallas import tpu as pltpu

_K = 8
_PB = 4


def _body(t_ref, a_ref, c_ref, x_hbm, e_hbm, n_hbm, x0_hbm, np_hbm, tg_hbm,
          xbuf, ebuf, nbuf, obuf, insem, outsem):
    def in_copies(k):
        s = k % 2
        d = pl.ds(k * _PB, _PB)
        return (
            pltpu.make_async_copy(x_hbm.at[d], xbuf.at[s], insem.at[0, s]),
            pltpu.make_async_copy(e_hbm.at[d], ebuf.at[s], insem.at[1, s]),
            pltpu.make_async_copy(n_hbm.at[d], nbuf.at[s], insem.at[2, s]),
        )

    def out_copies(k):
        s = k % 2
        d = pl.ds(k * _PB, _PB)
        return (
            pltpu.make_async_copy(obuf.at[s], x0_hbm.at[d], outsem.at[0, s]),
            pltpu.make_async_copy(ebuf.at[s], np_hbm.at[d], outsem.at[1, s]),
            pltpu.make_async_copy(nbuf.at[s], tg_hbm.at[d], outsem.at[2, s]),
        )

    for c in in_copies(0):
        c.start()
    for c in in_copies(1):
        c.start()

    for k in range(_K):
        s = k % 2
        for c in in_copies(k):
            c.wait()
        cx0, cnp, ctg = out_copies(k)
        cnp.start()
        ctg.start()
        if k >= 2:
            out_copies(k - 2)[0].wait()
        for j in range(_PB):
            ti = t_ref[k * _PB + j]
            obuf[s, j] = a_ref[ti] * xbuf[s, j] - c_ref[ti] * ebuf[s, j]
        cx0.start()
        if k + 2 < _K:
            cnp.wait()
            ctg.wait()
            for c in in_copies(k + 2):
                c.start()

    for k in (_K - 2, _K - 1):
        cx0, cnp, ctg = out_copies(k)
        cx0.wait()
        cnp.wait()
        ctg.wait()


def kernel(model_preds, x_t, x_0, noise, t,
           sqrt_recip_alphas_cumprod, sqrt_recipm1_alphas_cumprod):
    B, C, H, W = x_t.shape
    anyspec = pl.BlockSpec(memory_space=pl.ANY)
    grid_spec = pltpu.PrefetchScalarGridSpec(
        num_scalar_prefetch=3,
        grid=(1,),
        in_specs=[anyspec, anyspec, anyspec],
        out_specs=[anyspec, anyspec, anyspec],
        scratch_shapes=[
            pltpu.VMEM((2, _PB, C, H, W), jnp.float32),
            pltpu.VMEM((2, _PB, C, H, W), jnp.float32),
            pltpu.VMEM((2, _PB, C, H, W), jnp.float32),
            pltpu.VMEM((2, _PB, C, H, W), jnp.float32),
            pltpu.SemaphoreType.DMA((3, 2)),
            pltpu.SemaphoreType.DMA((3, 2)),
        ],
    )
    out = jax.ShapeDtypeStruct(x_t.shape, x_t.dtype)
    x0p, np_, tg = pl.pallas_call(
        _body,
        grid_spec=grid_spec,
        out_shape=[out, out, out],
    )(t, sqrt_recip_alphas_cumprod, sqrt_recipm1_alphas_cumprod,
      x_t, model_preds, noise)
    return (np_, x0p, tg)


# speedup vs baseline: 1.1291x; 1.1291x over previous
import jax
import jax.numpy as jnp
from jax.experimental import pallas as pl
from jax.experimental.pallas import tpu as pltpu

_K = 8
_PB = 4


def _body(t_ref, a_ref, c_ref, x_hbm, e_hbm, n_hbm, x0_hbm, np_hbm, tg_hbm,
          xbuf, ebuf, nbuf, obuf, insem, outsem):
    def in_copies(k):
        s = k % 3
        d = pl.ds(k * _PB, _PB)
        return (
            pltpu.make_async_copy(x_hbm.at[d], xbuf.at[s], insem.at[0, s]),
            pltpu.make_async_copy(e_hbm.at[d], ebuf.at[s], insem.at[1, s]),
            pltpu.make_async_copy(n_hbm.at[d], nbuf.at[s], insem.at[2, s]),
        )

    def out_copies(k):
        s = k % 3
        d = pl.ds(k * _PB, _PB)
        return (
            pltpu.make_async_copy(obuf.at[s], x0_hbm.at[d], outsem.at[0, s]),
            pltpu.make_async_copy(ebuf.at[s], np_hbm.at[d], outsem.at[1, s]),
            pltpu.make_async_copy(nbuf.at[s], tg_hbm.at[d], outsem.at[2, s]),
        )

    for c in in_copies(0):
        c.start()
    for c in in_copies(1):
        c.start()

    for k in range(_K):
        s = k % 3
        for c in in_copies(k):
            c.wait()
        cx0, cnp, ctg = out_copies(k)
        cnp.start()
        ctg.start()
        if k >= 3:
            out_copies(k - 3)[0].wait()
        for j in range(_PB):
            ti = t_ref[k * _PB + j]
            obuf[s, j] = a_ref[ti] * xbuf[s, j] - c_ref[ti] * ebuf[s, j]
        cx0.start()
        if k + 2 < _K:
            if k >= 1:
                _, pnp, ptg = out_copies(k - 1)
                pnp.wait()
                ptg.wait()
            for c in in_copies(k + 2):
                c.start()

    for k in (_K - 3, _K - 2, _K - 1):
        if k < 0:
            continue
        cx0, cnp, ctg = out_copies(k)
        cx0.wait()
        cnp.wait()
        ctg.wait()


def kernel(model_preds, x_t, x_0, noise, t,
           sqrt_recip_alphas_cumprod, sqrt_recipm1_alphas_cumprod):
    B, C, H, W = x_t.shape
    anyspec = pl.BlockSpec(memory_space=pl.ANY)
    grid_spec = pltpu.PrefetchScalarGridSpec(
        num_scalar_prefetch=3,
        grid=(1,),
        in_specs=[anyspec, anyspec, anyspec],
        out_specs=[anyspec, anyspec, anyspec],
        scratch_shapes=[
            pltpu.VMEM((3, _PB, C, H, W), jnp.float32),
            pltpu.VMEM((3, _PB, C, H, W), jnp.float32),
            pltpu.VMEM((3, _PB, C, H, W), jnp.float32),
            pltpu.VMEM((3, _PB, C, H, W), jnp.float32),
            pltpu.SemaphoreType.DMA((3, 3)),
            pltpu.SemaphoreType.DMA((3, 3)),
        ],
    )
    out = jax.ShapeDtypeStruct(x_t.shape, x_t.dtype)
    x0p, np_, tg = pl.pallas_call(
        _body,
        grid_spec=grid_spec,
        out_shape=[out, out, out],
    )(t, sqrt_recip_alphas_cumprod, sqrt_recipm1_alphas_cumprod,
      x_t, model_preds, noise)
    return (np_, x0p, tg)


# speedup vs baseline: 1.2588x; 1.1149x over previous
import jax
import jax.numpy as jnp
from jax.experimental import pallas as pl
from jax.experimental.pallas import tpu as pltpu

_K = 4
_PB = 8


def _body(t_ref, a_ref, c_ref, x_hbm, e_hbm, n_hbm, x0_hbm, np_hbm, tg_hbm,
          xbuf, ebuf, nbuf, obuf, insem, outsem):
    def in_copies(k):
        s = k % 3
        d = pl.ds(k * _PB, _PB)
        return (
            pltpu.make_async_copy(x_hbm.at[d], xbuf.at[s], insem.at[0, s]),
            pltpu.make_async_copy(e_hbm.at[d], ebuf.at[s], insem.at[1, s]),
            pltpu.make_async_copy(n_hbm.at[d], nbuf.at[s], insem.at[2, s]),
        )

    def out_copies(k):
        s = k % 3
        d = pl.ds(k * _PB, _PB)
        return (
            pltpu.make_async_copy(obuf.at[s], x0_hbm.at[d], outsem.at[0, s]),
            pltpu.make_async_copy(ebuf.at[s], np_hbm.at[d], outsem.at[1, s]),
            pltpu.make_async_copy(nbuf.at[s], tg_hbm.at[d], outsem.at[2, s]),
        )

    for c in in_copies(0):
        c.start()
    for c in in_copies(1):
        c.start()

    for k in range(_K):
        s = k % 3
        for c in in_copies(k):
            c.wait()
        cx0, cnp, ctg = out_copies(k)
        cnp.start()
        ctg.start()
        if k >= 3:
            out_copies(k - 3)[0].wait()
        for j in range(_PB):
            ti = t_ref[k * _PB + j]
            obuf[s, j] = a_ref[ti] * xbuf[s, j] - c_ref[ti] * ebuf[s, j]
        cx0.start()
        if k + 2 < _K:
            if k >= 1:
                _, pnp, ptg = out_copies(k - 1)
                pnp.wait()
                ptg.wait()
            for c in in_copies(k + 2):
                c.start()

    for k in (_K - 3, _K - 2, _K - 1):
        if k < 0:
            continue
        cx0, cnp, ctg = out_copies(k)
        cx0.wait()
        cnp.wait()
        ctg.wait()


def kernel(model_preds, x_t, x_0, noise, t,
           sqrt_recip_alphas_cumprod, sqrt_recipm1_alphas_cumprod):
    B, C, H, W = x_t.shape
    anyspec = pl.BlockSpec(memory_space=pl.ANY)
    grid_spec = pltpu.PrefetchScalarGridSpec(
        num_scalar_prefetch=3,
        grid=(1,),
        in_specs=[anyspec, anyspec, anyspec],
        out_specs=[anyspec, anyspec, anyspec],
        scratch_shapes=[
            pltpu.VMEM((3, _PB, C, H, W), jnp.float32),
            pltpu.VMEM((3, _PB, C, H, W), jnp.float32),
            pltpu.VMEM((3, _PB, C, H, W), jnp.float32),
            pltpu.VMEM((3, _PB, C, H, W), jnp.float32),
            pltpu.SemaphoreType.DMA((3, 3)),
            pltpu.SemaphoreType.DMA((3, 3)),
        ],
    )
    out = jax.ShapeDtypeStruct(x_t.shape, x_t.dtype)
    x0p, np_, tg = pl.pallas_call(
        _body,
        grid_spec=grid_spec,
        out_shape=[out, out, out],
    )(t, sqrt_recip_alphas_cumprod, sqrt_recipm1_alphas_cumprod,
      x_t, model_preds, noise)
    return (np_, x0p, tg)


# speedup vs baseline: 1.3995x; 1.1117x over previous
import jax
import jax.numpy as jnp
from jax.experimental import pallas as pl
from jax.experimental.pallas import tpu as pltpu

_PB = 16


def _body(t_ref, a_ref, c_ref, x_ref, eps_ref, nz_ref, x0_ref, np_ref, tg_ref):
    g = pl.program_id(0)
    np_ref[...] = eps_ref[...]
    tg_ref[...] = nz_ref[...]
    for j in range(_PB):
        ti = t_ref[g * _PB + j]
        x0_ref[j] = a_ref[ti] * x_ref[j] - c_ref[ti] * eps_ref[j]


def kernel(model_preds, x_t, x_0, noise, t,
           sqrt_recip_alphas_cumprod, sqrt_recipm1_alphas_cumprod):
    B, C, H, W = x_t.shape
    blk = pl.BlockSpec((_PB, C, H, W), lambda g, *_: (g, 0, 0, 0))
    grid_spec = pltpu.PrefetchScalarGridSpec(
        num_scalar_prefetch=3,
        grid=(B // _PB,),
        in_specs=[blk, blk, blk],
        out_specs=[blk, blk, blk],
    )
    out = jax.ShapeDtypeStruct(x_t.shape, x_t.dtype)
    x0p, np_, tg = pl.pallas_call(
        _body,
        grid_spec=grid_spec,
        out_shape=[out, out, out],
    )(t, sqrt_recip_alphas_cumprod, sqrt_recipm1_alphas_cumprod,
      x_t, model_preds, noise)
    return (np_, x0p, tg)
